# baseline (speedup 1.0000x reference)
import functools

import jax
import jax.numpy as jnp
from jax import lax
from jax.experimental import pallas as pl
from jax.experimental.pallas import tpu as pltpu
from jax.experimental.pallas import tpu_sc as plsc

_L = 16


@functools.cache
def _build_sc_call(B, F, V, D):
    info = plsc.get_sparse_core_info()
    NC, NS = info.num_cores, info.num_subcores
    NW = NC * NS
    assert B % NW == 0 and D == _L and (F - 1) % _L == 0
    b_per_w = B // NW
    OUT_D = D + F - 1
    NF = (F - 1) // _L

    mesh = plsc.VectorSubcoreMesh(core_axis_name="c", subcore_axis_name="s")

    @functools.partial(
        pl.kernel,
        mesh=mesh,
        compiler_params=pltpu.CompilerParams(
            use_tc_tiling_on_sc=False, needs_layout_passes=False),
        out_type=jax.ShapeDtypeStruct((B, OUT_D), jnp.float32),
        scratch_types=[
            pltpu.VMEM((b_per_w,), jnp.int32),
            pltpu.VMEM((b_per_w, F), jnp.float32),
            pltpu.VMEM((b_per_w, D), jnp.float32),
            pltpu.VMEM((b_per_w, OUT_D), jnp.float32),
            pltpu.SemaphoreType.DMA,
        ],
    )
    def sc_kernel(x_hbm, table_hbm, out_hbm, idx_v, x_v, emb_v, out_v, sem):
        wid = lax.axis_index("s") * NC + lax.axis_index("c")
        base = wid * b_per_w
        pltpu.sync_copy(x_hbm.at[pl.ds(base, b_per_w)], x_v)

        lanes = lax.iota(jnp.int32, _L)
        zeros = jnp.zeros((_L,), jnp.int32)

        def idx_body(i, _):
            rows = lanes + i * _L
            ids = plsc.load_gather(x_v, [rows, zeros])
            idx_v[pl.ds(i * _L, _L)] = ids.astype(jnp.int32)
            return 0

        lax.fori_loop(0, b_per_w // _L, idx_body, 0, unroll=4)

        gather = pltpu.async_copy(table_hbm.at[idx_v], emb_v, sem)

        def feat_body(r, _):
            rr = jnp.full((_L,), r, jnp.int32)
            for j in range(NF):
                v = plsc.load_gather(x_v, [rr, lanes + (1 + j * _L)])
                out_v[r, pl.ds(D + j * _L, _L)] = v
            return 0

        lax.fori_loop(0, b_per_w, feat_body, 0, unroll=4)
        gather.wait()

        def emb_body(r, _):
            out_v[r, pl.ds(0, D)] = emb_v[r, :]
            return 0

        lax.fori_loop(0, b_per_w, emb_body, 0, unroll=4)
        pltpu.sync_copy(out_v, out_hbm.at[pl.ds(base, b_per_w)])

    return sc_kernel


def kernel(x, table):
    B, F = x.shape
    V, D = table.shape
    return _build_sc_call(B, F, V, D)(x, table)


# speedup vs baseline: 5.4035x; 5.4035x over previous
import functools

import jax
import jax.numpy as jnp
from jax import lax
from jax.experimental import pallas as pl
from jax.experimental.pallas import tpu as pltpu
from jax.experimental.pallas import tpu_sc as plsc

_L = 16


@functools.cache
def _build_sc_call(B, F, V, D):
    info = plsc.get_sparse_core_info()
    NC, NS = info.num_cores, info.num_subcores
    NW = NC * NS
    assert B % NW == 0 and D == _L
    b_per_w = B // NW
    OUT_D = D + F - 1
    TR = D // 8
    K = 16
    NB = b_per_w // K
    NP = b_per_w // _L

    mesh = plsc.VectorSubcoreMesh(core_axis_name="c", subcore_axis_name="s")

    @functools.partial(
        pl.kernel,
        mesh=mesh,
        compiler_params=pltpu.CompilerParams(
            use_tc_tiling_on_sc=True, needs_layout_passes=False),
        out_type=jax.ShapeDtypeStruct((OUT_D, B), jnp.float32),
        scratch_types=[
            pltpu.VMEM((F, b_per_w), jnp.float32),
            pltpu.VMEM((OUT_D, b_per_w), jnp.float32),
            pltpu.VMEM((K, TR, 8, 128), jnp.float32),
            pltpu.VMEM((b_per_w,), jnp.int32),
            pltpu.SemaphoreType.DMA,
        ],
    )
    def sc_kernel(x_hbm, t_hbm, out_hbm, xv, outv, slots, idv, sem):
        wid = lax.axis_index("s") * NC + lax.axis_index("c")
        base = wid * b_per_w
        pltpu.sync_copy(x_hbm.at[:, pl.ds(base, b_per_w)], xv)

        lanes = lax.iota(jnp.int32, _L)
        arow = lanes >> 3
        brow = lanes & 7

        for t in range(NP):
            idv[pl.ds(t * _L, _L)] = xv[0, pl.ds(t * _L, _L)].astype(jnp.int32)

        def batch_body(bi, _):
            sids = idv[pl.ds(bi * K, K)]
            copies = []
            for k in range(K):
                sid = sids[k]
                col = pl.multiple_of((sid >> 7) * 128, 128)
                copies.append(pltpu.async_copy(
                    t_hbm.at[:, :, pl.ds(col, 128)], slots.at[k], sem))
            for t in range(NP):
                outv[bi + D, pl.ds(t * _L, _L)] = xv[bi + 1, pl.ds(t * _L, _L)]
            for k in range(K):
                copies[k].wait()
                j = bi * K + k
                sid = sids[k]
                lane = jnp.full((_L,), sid & 127, jnp.int32)
                vals = plsc.load_gather(slots.at[k], [arow, brow, lane])
                plsc.store_scatter(outv, [lanes, jnp.full((_L,), j, jnp.int32)],
                                   vals)
            return 0

        lax.fori_loop(0, NB, batch_body, 0)
        pltpu.sync_copy(outv, out_hbm.at[:, pl.ds(base, b_per_w)])

    return sc_kernel


def kernel(x, table):
    B, F = x.shape
    V, D = table.shape
    outT = _build_sc_call(B, F, V, D)(x.T, table.T.reshape(D // 8, 8, V))
    return outT.T


# speedup vs baseline: 5.9617x; 1.1033x over previous
import functools

import jax
import jax.numpy as jnp
from jax import lax
from jax.experimental import pallas as pl
from jax.experimental.pallas import tpu as pltpu
from jax.experimental.pallas import tpu_sc as plsc

_L = 16


@functools.cache
def _build_sc_call(B, F, V, D):
    info = plsc.get_sparse_core_info()
    NC, NS = info.num_cores, info.num_subcores
    NW = NC * NS
    assert B % NW == 0 and D == _L
    b_per_w = B // NW
    OUT_D = D + F - 1
    TR = D // 8
    K = 16
    NB = b_per_w // K
    NP = b_per_w // _L

    mesh = plsc.VectorSubcoreMesh(core_axis_name="c", subcore_axis_name="s")

    @functools.partial(
        pl.kernel,
        mesh=mesh,
        compiler_params=pltpu.CompilerParams(
            use_tc_tiling_on_sc=True, needs_layout_passes=False),
        out_type=jax.ShapeDtypeStruct((OUT_D, B), jnp.float32),
        scratch_types=[
            pltpu.VMEM((F, b_per_w), jnp.float32),
            pltpu.VMEM((OUT_D, b_per_w), jnp.float32),
            pltpu.VMEM((2 * K, TR, 8, 128), jnp.float32),
            pltpu.VMEM((b_per_w,), jnp.int32),
            pltpu.SemaphoreType.DMA((2,)),
        ],
    )
    def sc_kernel(x_hbm, t_hbm, out_hbm, xv, outv, slots, idv, sems):
        wid = lax.axis_index("s") * NC + lax.axis_index("c")
        base = wid * b_per_w
        pltpu.sync_copy(x_hbm.at[:, pl.ds(base, b_per_w)], xv)

        lanes = lax.iota(jnp.int32, _L)
        arow = lanes >> 3
        brow = lanes & 7

        for t in range(NP):
            idv[pl.ds(t * _L, _L)] = xv[0, pl.ds(t * _L, _L)].astype(jnp.int32)

        def fire(bi, g):
            sids = idv[pl.ds(bi * K, K)]
            cols = (sids >> 7) * 128
            for k in range(K):
                col = pl.multiple_of(cols[k], 128)
                pltpu.async_copy(t_hbm.at[:, :, pl.ds(col, 128)],
                                 slots.at[g * K + k], sems.at[g])

        def drain_extract(bi, g):
            sids = idv[pl.ds(bi * K, K)]
            lns = sids & 127
            for k in range(K):
                pltpu.make_async_copy(t_hbm.at[:, :, pl.ds(0, 128)],
                                      slots.at[g * K + k], sems.at[g]).wait()
            for k in range(K):
                lane = jnp.full((_L,), lns[k], jnp.int32)
                vals = plsc.load_gather(slots.at[g * K + k], [arow, brow, lane])
                plsc.store_scatter(
                    outv, [lanes, jnp.full((_L,), bi * K + k, jnp.int32)], vals)

        def feat_row(r):
            for t in range(NP):
                outv[r + D - 1, pl.ds(t * _L, _L)] = xv[r, pl.ds(t * _L, _L)]

        fire(0, 0)

        def pair_body(h, _):
            b1 = 2 * h + 1
            fire(b1, 1)
            drain_extract(2 * h, 0)
            feat_row(b1)

            @pl.when(h < NB // 2 - 1)
            def _():
                fire(b1 + 1, 0)

            drain_extract(b1, 1)
            feat_row(b1 + 1)
            return 0

        lax.fori_loop(0, NB // 2, pair_body, 0)
        pltpu.sync_copy(outv, out_hbm.at[:, pl.ds(base, b_per_w)])

    return sc_kernel


def kernel(x, table):
    B, F = x.shape
    V, D = table.shape
    outT = _build_sc_call(B, F, V, D)(x.T, table.T.reshape(D // 8, 8, V))
    return outT.T
